# Initial kernel scaffold; baseline (speedup 1.0000x reference)
import jax
import jax.numpy as jnp
from jax.experimental import pallas as pl


def kernel(x, edge_index, edge_weight, batch_vec, W1_l, b1_l, W1_r, W2_l, b2_l, W2_r):
    raise NotImplementedError("write your pallas kernel here")


# speedup vs baseline: 3.6922x; 3.6922x over previous
import jax
import jax.numpy as jnp
from jax import lax
from jax.experimental import pallas as pl
from jax.experimental.pallas import tpu as pltpu
from jax.experimental.pallas import tpu_sc as plsc

_N = 10000
_E = 320000
_G = 64
_HID = 64
_OUT = 128

_NC, _NS = 2, 16
_NW = _NC * _NS
_CHE = 4000
_NST = _CHE // 16
_NCH = _E // _CHE
_QCH = _NCH // 4


def _make_sc_agg(with_count):
  mesh = plsc.VectorSubcoreMesh(core_axis_name="c", subcore_axis_name="s",
                                num_cores=_NC, num_subcores=_NS)
  out_type = [jax.ShapeDtypeStruct((_HID * _N,), jnp.float32)]
  scratch = [
      pltpu.VMEM((_N,), jnp.float32),
      pltpu.VMEM((_N,), jnp.float32),
      pltpu.VMEM((_N,), jnp.float32),
      pltpu.VMEM((_N,), jnp.float32),
      pltpu.VMEM((_CHE,), jnp.int32),
      pltpu.VMEM((_CHE,), jnp.int32),
  ]
  if with_count:
    out_type.append(jax.ShapeDtypeStruct((4 * _N,), jnp.float32))
    scratch.append(pltpu.VMEM((_N,), jnp.float32))

  def body(*refs):
    if with_count:
      (tab_hbm, src_hbm, dst_hbm, agg_out, cnt_out,
       tabA, tabB, accA, accB, schunk, dchunk, accC) = refs
    else:
      (tab_hbm, src_hbm, dst_hbm, agg_out,
       tabA, tabB, accA, accB, schunk, dchunk) = refs
    c = lax.axis_index("c")
    s = lax.axis_index("s")
    t = c * _NS + s
    q = t - (_NW - 4)

    pltpu.sync_copy(tab_hbm.at[pl.ds((2 * t) * _N, _N)], tabA)
    pltpu.sync_copy(tab_hbm.at[pl.ds((2 * t + 1) * _N, _N)], tabB)

    def zero(i, carry):
      z = jnp.zeros((16,), jnp.float32)
      accA[pl.ds(i * 16, 16)] = z
      accB[pl.ds(i * 16, 16)] = z
      if with_count:
        accC[pl.ds(i * 16, 16)] = z
      return carry

    lax.fori_loop(0, _N // 16, zero, 0)

    def chunk(jc, carry):
      b = jc * _CHE
      pltpu.sync_copy(src_hbm.at[pl.ds(b, _CHE)], schunk)
      pltpu.sync_copy(dst_hbm.at[pl.ds(b, _CHE)], dchunk)

      def step(i, carry2):
        svec = schunk[pl.ds(i * 16, 16)]
        dvec = dchunk[pl.ds(i * 16, 16)]
        plsc.addupdate_scatter(accA, [dvec], plsc.load_gather(tabA, [svec]))
        plsc.addupdate_scatter(accB, [dvec], plsc.load_gather(tabB, [svec]))
        return carry2

      lax.fori_loop(0, _NST, step, 0)

      if with_count:
        @pl.when((q >= 0) & (jc // _QCH == q))
        def _():
          def cstep(i, carry2):
            dvec = dchunk[pl.ds(i * 16, 16)]
            plsc.addupdate_scatter(accC, [dvec],
                                   jnp.full((16,), 1.0, jnp.float32))
            return carry2
          lax.fori_loop(0, _NST, cstep, 0)
      return carry

    lax.fori_loop(0, _NCH, chunk, 0)

    pltpu.sync_copy(accA, agg_out.at[pl.ds((2 * t) * _N, _N)])
    pltpu.sync_copy(accB, agg_out.at[pl.ds((2 * t + 1) * _N, _N)])
    if with_count:
      @pl.when(q >= 0)
      def _():
        pltpu.sync_copy(accC, cnt_out.at[pl.ds(q * _N, _N)])

  return pl.kernel(
      body, out_type=out_type, mesh=mesh, scratch_types=scratch,
      compiler_params=pltpu.CompilerParams(needs_layout_passes=False))


_sc_agg_cache = {}


def _sc_agg(with_count):
  if with_count not in _sc_agg_cache:
    _sc_agg_cache[with_count] = _make_sc_agg(with_count)
  return _sc_agg_cache[with_count]


def _proj_body(x_ref, wl_ref, wr_ref, p_ref, r_ref):
  x = x_ref[...]
  dn = (((1,), (1,)), ((), ()))
  p_ref[...] = lax.dot_general(wl_ref[...], x, dn,
                               preferred_element_type=jnp.float32)
  r_ref[...] = lax.dot_general(wr_ref[...], x, dn,
                               preferred_element_type=jnp.float32)


def _l1_body(agg_ref, cnt_ref, xr_ref, b_ref, o_ref):
  cnt = cnt_ref[0] + cnt_ref[1] + cnt_ref[2] + cnt_ref[3]
  rc = 1.0 / jnp.maximum(cnt, 1.0)
  o_ref[...] = jnp.maximum(agg_ref[...] * rc + b_ref[...][:, 0:1] + xr_ref[...],
                           0.0)


def _l2_body(agg_ref, cnt_ref, h1_ref, wl_ref, b_ref, wr_ref, bv_ref,
             ones_ref, o_ref):
  cnt = cnt_ref[0] + cnt_ref[1] + cnt_ref[2] + cnt_ref[3]
  rc = 1.0 / jnp.maximum(cnt, 1.0)
  mean = agg_ref[...] * rc
  dn = (((1,), (0,)), ((), ()))
  h2 = lax.dot_general(wl_ref[...], mean, dn,
                       preferred_element_type=jnp.float32)
  h2 = h2 + b_ref[...][:, 0:1] + lax.dot_general(
      wr_ref[...], h1_ref[...], dn, preferred_element_type=jnp.float32)
  h2 = jnp.maximum(h2, 0.0)
  oh = (bv_ref[...] == lax.broadcasted_iota(jnp.int32, (_N, _G), 1))
  oh = oh.astype(jnp.float32)
  pooled = lax.dot_general(oh, h2, (((0,), (1,)), ((), ())),
                           preferred_element_type=jnp.float32)
  cg = lax.dot_general(oh, ones_ref[...], (((0,), (0,)), ((), ())),
                       preferred_element_type=jnp.float32)
  o_ref[...] = pooled * (1.0 / jnp.maximum(cg[:, 0:1], 1.0))


_tc_proj = pl.pallas_call(
    _proj_body,
    out_shape=[jax.ShapeDtypeStruct((_HID, _N), jnp.float32),
               jax.ShapeDtypeStruct((_HID, _N), jnp.float32)])

_tc_l1 = pl.pallas_call(
    _l1_body,
    out_shape=jax.ShapeDtypeStruct((_HID, _N), jnp.float32))

_tc_l2 = pl.pallas_call(
    _l2_body,
    out_shape=jax.ShapeDtypeStruct((_G, _OUT), jnp.float32))


def kernel(x, edge_index, edge_weight, batch_vec, W1_l, b1_l, W1_r,
           W2_l, b2_l, W2_r):
  del edge_weight
  f32 = jnp.float32
  src = edge_index[0]
  dst = edge_index[1]

  p1T, xrT = _tc_proj(x, W1_l, W1_r)

  agg1T, cnt4 = _sc_agg(True)(p1T.reshape(-1), src, dst)
  agg1T = agg1T.reshape(_HID, _N)
  cnt4 = cnt4.reshape(4, _N)

  b1b = jnp.broadcast_to(b1_l[:, None], (_HID, 128)).astype(f32)
  h1T = _tc_l1(agg1T, cnt4, xrT, b1b)

  (agg2T,) = _sc_agg(False)(h1T.reshape(-1), src, dst)
  agg2T = agg2T.reshape(_HID, _N)

  b2b = jnp.broadcast_to(b2_l[:, None], (_OUT, 128)).astype(f32)
  out = _tc_l2(agg2T, cnt4, h1T, W2_l, b2b, W2_r,
               batch_vec.reshape(_N, 1), jnp.ones((_N, 8), f32))
  return out


# speedup vs baseline: 4.9669x; 1.3452x over previous
import jax
import jax.numpy as jnp
from jax import lax
from jax.experimental import pallas as pl
from jax.experimental.pallas import tpu as pltpu
from jax.experimental.pallas import tpu_sc as plsc

_N = 10000
_E = 320000
_G = 64
_HID = 64
_OUT = 128

_NC, _NS = 2, 16
_PPT = 4
_EH = _E // _NC
_CHE = 8000
_NST = _CHE // 16
_UNR = 4
_NCH = _EH // _CHE
_QLOC = _NCH // 2


def _make_sc_agg(with_count):
  mesh = plsc.VectorSubcoreMesh(core_axis_name="c", subcore_axis_name="s",
                                num_cores=_NC, num_subcores=_NS)
  out_type = [jax.ShapeDtypeStruct((_NC * _HID * _N,), jnp.float32)]
  scratch = (
      [pltpu.VMEM((_N,), jnp.float32) for _ in range(_PPT)]
      + [pltpu.VMEM((_N,), jnp.float32) for _ in range(_PPT)]
      + [pltpu.VMEM((_CHE,), jnp.int32),
         pltpu.VMEM((_CHE,), jnp.int32)]
  )
  if with_count:
    out_type.append(jax.ShapeDtypeStruct((4 * _N,), jnp.float32))
    scratch.append(pltpu.VMEM((_N,), jnp.float32))

  def body(*refs):
    if with_count:
      (tab_hbm, src_hbm, dst_hbm, agg_out, cnt_out) = refs[:5]
      rest = refs[5:]
    else:
      (tab_hbm, src_hbm, dst_hbm, agg_out) = refs[:4]
      rest = refs[4:]
    tabs = rest[:_PPT]
    accs = rest[_PPT:2 * _PPT]
    schunk, dchunk = rest[2 * _PPT:2 * _PPT + 2]
    accC = rest[2 * _PPT + 2] if with_count else None

    c = lax.axis_index("c")
    s = lax.axis_index("s")

    for u in range(_PPT):
      pltpu.sync_copy(tab_hbm.at[pl.ds((_PPT * s + u) * _N, _N)], tabs[u])

    def zero(i, carry):
      z = jnp.zeros((16,), jnp.float32)
      for u in range(_PPT):
        accs[u][pl.ds(i * 16, 16)] = z
      if with_count:
        accC[pl.ds(i * 16, 16)] = z
      return carry

    lax.fori_loop(0, _N // 16, zero, 0)

    ebase = c * _EH

    def chunk(jc, carry):
      b = ebase + jc * _CHE
      pltpu.sync_copy(src_hbm.at[pl.ds(b, _CHE)], schunk)
      pltpu.sync_copy(dst_hbm.at[pl.ds(b, _CHE)], dchunk)

      def step(i, carry2):
        for u2 in range(_UNR):
          o = i * (16 * _UNR) + u2 * 16
          svec = schunk[pl.ds(o, 16)]
          dvec = dchunk[pl.ds(o, 16)]
          for u in range(_PPT):
            plsc.addupdate_scatter(accs[u], [dvec],
                                   plsc.load_gather(tabs[u], [svec]))
        return carry2

      lax.fori_loop(0, _NST // _UNR, step, 0)

      if with_count:
        @pl.when((s >= _NS - 2) & (jc // _QLOC == s - (_NS - 2)))
        def _():
          def cstep(i, carry2):
            for u2 in range(_UNR):
              o = i * (16 * _UNR) + u2 * 16
              dvec = dchunk[pl.ds(o, 16)]
              plsc.addupdate_scatter(accC, [dvec],
                                     jnp.full((16,), 1.0, jnp.float32))
            return carry2
          lax.fori_loop(0, _NST // _UNR, cstep, 0)
      return carry

    lax.fori_loop(0, _NCH, chunk, 0)

    for u in range(_PPT):
      pltpu.sync_copy(accs[u],
                      agg_out.at[pl.ds((c * _HID + _PPT * s + u) * _N, _N)])
    if with_count:
      @pl.when(s >= _NS - 2)
      def _():
        q = c * 2 + (s - (_NS - 2))
        pltpu.sync_copy(accC, cnt_out.at[pl.ds(q * _N, _N)])

  return pl.kernel(
      body, out_type=out_type, mesh=mesh, scratch_types=scratch,
      compiler_params=pltpu.CompilerParams(needs_layout_passes=False))


_sc_agg_cache = {}


def _sc_agg(with_count):
  if with_count not in _sc_agg_cache:
    _sc_agg_cache[with_count] = _make_sc_agg(with_count)
  return _sc_agg_cache[with_count]


def _proj_body(x_ref, wl_ref, wr_ref, p_ref, r_ref):
  x = x_ref[...]
  dn = (((1,), (1,)), ((), ()))
  p_ref[...] = lax.dot_general(wl_ref[...], x, dn,
                               preferred_element_type=jnp.float32)
  r_ref[...] = lax.dot_general(wr_ref[...], x, dn,
                               preferred_element_type=jnp.float32)


def _l1_body(agg_ref, cnt_ref, xr_ref, b_ref, o_ref):
  cnt = cnt_ref[0] + cnt_ref[1] + cnt_ref[2] + cnt_ref[3]
  rc = 1.0 / jnp.maximum(cnt, 1.0)
  agg = agg_ref[0] + agg_ref[1]
  o_ref[...] = jnp.maximum(agg * rc + b_ref[...][:, 0:1] + xr_ref[...], 0.0)


def _l2_body(agg_ref, cnt_ref, h1_ref, wl_ref, b_ref, wr_ref, bv_ref,
             ones_ref, o_ref):
  cnt = cnt_ref[0] + cnt_ref[1] + cnt_ref[2] + cnt_ref[3]
  rc = 1.0 / jnp.maximum(cnt, 1.0)
  mean = (agg_ref[0] + agg_ref[1]) * rc
  dn = (((1,), (0,)), ((), ()))
  h2 = lax.dot_general(wl_ref[...], mean, dn,
                       preferred_element_type=jnp.float32)
  h2 = h2 + b_ref[...][:, 0:1] + lax.dot_general(
      wr_ref[...], h1_ref[...], dn, preferred_element_type=jnp.float32)
  h2 = jnp.maximum(h2, 0.0)
  oh = (bv_ref[...] == lax.broadcasted_iota(jnp.int32, (_N, _G), 1))
  oh = oh.astype(jnp.float32)
  pooled = lax.dot_general(oh, h2, (((0,), (1,)), ((), ())),
                           preferred_element_type=jnp.float32)
  cg = lax.dot_general(oh, ones_ref[...], (((0,), (0,)), ((), ())),
                       preferred_element_type=jnp.float32)
  o_ref[...] = pooled * (1.0 / jnp.maximum(cg[:, 0:1], 1.0))


_tc_proj = pl.pallas_call(
    _proj_body,
    out_shape=[jax.ShapeDtypeStruct((_HID, _N), jnp.float32),
               jax.ShapeDtypeStruct((_HID, _N), jnp.float32)])

_tc_l1 = pl.pallas_call(
    _l1_body,
    out_shape=jax.ShapeDtypeStruct((_HID, _N), jnp.float32))

_tc_l2 = pl.pallas_call(
    _l2_body,
    out_shape=jax.ShapeDtypeStruct((_G, _OUT), jnp.float32))


def kernel(x, edge_index, edge_weight, batch_vec, W1_l, b1_l, W1_r,
           W2_l, b2_l, W2_r):
  del edge_weight
  f32 = jnp.float32
  src = edge_index[0]
  dst = edge_index[1]

  p1T, xrT = _tc_proj(x, W1_l, W1_r)

  agg1T, cnt4 = _sc_agg(True)(p1T.reshape(-1), src, dst)
  agg1T = agg1T.reshape(_NC, _HID, _N)
  cnt4 = cnt4.reshape(4, _N)

  b1b = jnp.broadcast_to(b1_l[:, None], (_HID, 128)).astype(f32)
  h1T = _tc_l1(agg1T, cnt4, xrT, b1b)

  (agg2T,) = _sc_agg(False)(h1T.reshape(-1), src, dst)
  agg2T = agg2T.reshape(_NC, _HID, _N)

  b2b = jnp.broadcast_to(b2_l[:, None], (_OUT, 128)).astype(f32)
  out = _tc_l2(agg2T, cnt4, h1T, W2_l, b2b, W2_r,
               batch_vec.reshape(_N, 1), jnp.ones((_N, 8), f32))
  return out


# speedup vs baseline: 9.4472x; 1.9020x over previous
import jax
import jax.numpy as jnp
from jax import lax
from jax.experimental import pallas as pl
from jax.experimental.pallas import tpu as pltpu
from jax.experimental.pallas import tpu_sc as plsc

_N = 10000
_E = 320000
_G = 64
_HID = 64
_OUT = 128

_NC, _NS = 2, 16
_PPT = 4
_EH = _E // _NC
_CHE = 8000
_NST = _CHE // 16
_UNR = 4
_NCH = _EH // _CHE
_QLOC = _NCH // 2


def _make_sc_agg(with_count):
  mesh = plsc.VectorSubcoreMesh(core_axis_name="c", subcore_axis_name="s",
                                num_cores=_NC, num_subcores=_NS)
  out_type = [jax.ShapeDtypeStruct((_NC * _HID * _N,), jnp.float32)]
  scratch = (
      [pltpu.VMEM((_N,), jnp.float32) for _ in range(_PPT)]
      + [pltpu.VMEM((_N,), jnp.float32) for _ in range(_PPT)]
      + [pltpu.VMEM((_CHE,), jnp.int32),
         pltpu.VMEM((_CHE,), jnp.int32)]
  )
  if with_count:
    out_type.append(jax.ShapeDtypeStruct((4 * _N,), jnp.float32))
    scratch.append(pltpu.VMEM((_N,), jnp.float32))

  def body(*refs):
    if with_count:
      (tab_hbm, src_hbm, dst_hbm, agg_out, cnt_out) = refs[:5]
      rest = refs[5:]
    else:
      (tab_hbm, src_hbm, dst_hbm, agg_out) = refs[:4]
      rest = refs[4:]
    tabs = rest[:_PPT]
    accs = rest[_PPT:2 * _PPT]
    schunk, dchunk = rest[2 * _PPT:2 * _PPT + 2]
    accC = rest[2 * _PPT + 2] if with_count else None

    c = lax.axis_index("c")
    s = lax.axis_index("s")

    for u in range(_PPT):
      pltpu.sync_copy(tab_hbm.at[pl.ds((_PPT * s + u) * _N, _N)], tabs[u])

    @plsc.parallel_loop(0, _N // 16, unroll=4)
    def zero(i):
      z = jnp.zeros((16,), jnp.float32)
      for u in range(_PPT):
        accs[u][pl.ds(i * 16, 16)] = z
      if with_count:
        accC[pl.ds(i * 16, 16)] = z

    ebase = c * _EH

    def chunk(jc, carry):
      b = ebase + jc * _CHE
      pltpu.sync_copy(src_hbm.at[pl.ds(b, _CHE)], schunk)
      pltpu.sync_copy(dst_hbm.at[pl.ds(b, _CHE)], dchunk)

      @plsc.parallel_loop(0, _NST, unroll=_UNR)
      def step(i):
        svec = schunk[pl.ds(i * 16, 16)]
        dvec = dchunk[pl.ds(i * 16, 16)]
        for u in range(_PPT):
          plsc.addupdate_scatter(accs[u], [dvec],
                                 plsc.load_gather(tabs[u], [svec]))

      if with_count:
        @pl.when((s >= _NS - 2) & (jc // _QLOC == s - (_NS - 2)))
        def _():
          @plsc.parallel_loop(0, _NST, unroll=_UNR)
          def cstep(i):
            dvec = dchunk[pl.ds(i * 16, 16)]
            plsc.addupdate_scatter(accC, [dvec],
                                   jnp.full((16,), 1.0, jnp.float32))
      return carry

    lax.fori_loop(0, _NCH, chunk, 0)

    for u in range(_PPT):
      pltpu.sync_copy(accs[u],
                      agg_out.at[pl.ds((c * _HID + _PPT * s + u) * _N, _N)])
    if with_count:
      @pl.when(s >= _NS - 2)
      def _():
        q = c * 2 + (s - (_NS - 2))
        pltpu.sync_copy(accC, cnt_out.at[pl.ds(q * _N, _N)])

  return pl.kernel(
      body, out_type=out_type, mesh=mesh, scratch_types=scratch,
      compiler_params=pltpu.CompilerParams(needs_layout_passes=False))


_sc_agg_cache = {}


def _sc_agg(with_count):
  if with_count not in _sc_agg_cache:
    _sc_agg_cache[with_count] = _make_sc_agg(with_count)
  return _sc_agg_cache[with_count]


def _proj_body(x_ref, wl_ref, wr_ref, p_ref, r_ref):
  x = x_ref[...]
  dn = (((1,), (1,)), ((), ()))
  p_ref[...] = lax.dot_general(wl_ref[...], x, dn,
                               preferred_element_type=jnp.float32)
  r_ref[...] = lax.dot_general(wr_ref[...], x, dn,
                               preferred_element_type=jnp.float32)


def _l1_body(agg_ref, cnt_ref, xr_ref, b_ref, o_ref):
  cnt = cnt_ref[0] + cnt_ref[1] + cnt_ref[2] + cnt_ref[3]
  rc = 1.0 / jnp.maximum(cnt, 1.0)
  agg = agg_ref[0] + agg_ref[1]
  o_ref[...] = jnp.maximum(agg * rc + b_ref[...][:, 0:1] + xr_ref[...], 0.0)


def _l2_body(agg_ref, cnt_ref, h1_ref, wl_ref, b_ref, wr_ref, bv_ref,
             ones_ref, o_ref):
  cnt = cnt_ref[0] + cnt_ref[1] + cnt_ref[2] + cnt_ref[3]
  rc = 1.0 / jnp.maximum(cnt, 1.0)
  mean = (agg_ref[0] + agg_ref[1]) * rc
  dn = (((1,), (0,)), ((), ()))
  h2 = lax.dot_general(wl_ref[...], mean, dn,
                       preferred_element_type=jnp.float32)
  h2 = h2 + b_ref[...][:, 0:1] + lax.dot_general(
      wr_ref[...], h1_ref[...], dn, preferred_element_type=jnp.float32)
  h2 = jnp.maximum(h2, 0.0)
  oh = (bv_ref[...] == lax.broadcasted_iota(jnp.int32, (_N, _G), 1))
  oh = oh.astype(jnp.float32)
  pooled = lax.dot_general(oh, h2, (((0,), (1,)), ((), ())),
                           preferred_element_type=jnp.float32)
  cg = lax.dot_general(oh, ones_ref[...], (((0,), (0,)), ((), ())),
                       preferred_element_type=jnp.float32)
  o_ref[...] = pooled * (1.0 / jnp.maximum(cg[:, 0:1], 1.0))


_tc_proj = pl.pallas_call(
    _proj_body,
    out_shape=[jax.ShapeDtypeStruct((_HID, _N), jnp.float32),
               jax.ShapeDtypeStruct((_HID, _N), jnp.float32)])

_tc_l1 = pl.pallas_call(
    _l1_body,
    out_shape=jax.ShapeDtypeStruct((_HID, _N), jnp.float32))

_tc_l2 = pl.pallas_call(
    _l2_body,
    out_shape=jax.ShapeDtypeStruct((_G, _OUT), jnp.float32))


def kernel(x, edge_index, edge_weight, batch_vec, W1_l, b1_l, W1_r,
           W2_l, b2_l, W2_r):
  del edge_weight
  f32 = jnp.float32
  src = edge_index[0]
  dst = edge_index[1]

  p1T, xrT = _tc_proj(x, W1_l, W1_r)

  agg1T, cnt4 = _sc_agg(True)(p1T.reshape(-1), src, dst)
  agg1T = agg1T.reshape(_NC, _HID, _N)
  cnt4 = cnt4.reshape(4, _N)

  b1b = jnp.broadcast_to(b1_l[:, None], (_HID, 128)).astype(f32)
  h1T = _tc_l1(agg1T, cnt4, xrT, b1b)

  (agg2T,) = _sc_agg(False)(h1T.reshape(-1), src, dst)
  agg2T = agg2T.reshape(_NC, _HID, _N)

  b2b = jnp.broadcast_to(b2_l[:, None], (_OUT, 128)).astype(f32)
  out = _tc_l2(agg2T, cnt4, h1T, W2_l, b2b, W2_r,
               batch_vec.reshape(_N, 1), jnp.ones((_N, 8), f32))
  return out


# speedup vs baseline: 9.9820x; 1.0566x over previous
import jax
import jax.numpy as jnp
from jax import lax
from jax.experimental import pallas as pl
from jax.experimental.pallas import tpu as pltpu
from jax.experimental.pallas import tpu_sc as plsc

_N = 10000
_E = 320000
_G = 64
_HID = 64
_OUT = 128

_NC, _NS = 2, 16
_PPT = 4
_EH = _E // _NC
_CHE = 16000
_NST = _CHE // 16
_UNR = 8
_NCH = _EH // _CHE
_QLOC = _NCH // 2


def _make_sc_agg(with_count):
  mesh = plsc.VectorSubcoreMesh(core_axis_name="c", subcore_axis_name="s",
                                num_cores=_NC, num_subcores=_NS)
  out_type = [jax.ShapeDtypeStruct((_NC * _HID * _N,), jnp.float32)]
  scratch = (
      [pltpu.VMEM((_N,), jnp.float32) for _ in range(_PPT)]
      + [pltpu.VMEM((_N,), jnp.float32) for _ in range(_PPT)]
      + [pltpu.VMEM((_CHE,), jnp.int32),
         pltpu.VMEM((_CHE,), jnp.int32)]
  )
  if with_count:
    out_type.append(jax.ShapeDtypeStruct((4 * _N,), jnp.float32))
    scratch.append(pltpu.VMEM((_N,), jnp.float32))

  def body(*refs):
    if with_count:
      (tab_hbm, src_hbm, dst_hbm, agg_out, cnt_out) = refs[:5]
      rest = refs[5:]
    else:
      (tab_hbm, src_hbm, dst_hbm, agg_out) = refs[:4]
      rest = refs[4:]
    tabs = rest[:_PPT]
    accs = rest[_PPT:2 * _PPT]
    schunk, dchunk = rest[2 * _PPT:2 * _PPT + 2]
    accC = rest[2 * _PPT + 2] if with_count else None

    c = lax.axis_index("c")
    s = lax.axis_index("s")

    for u in range(_PPT):
      pltpu.sync_copy(tab_hbm.at[pl.ds((_PPT * s + u) * _N, _N)], tabs[u])

    @plsc.parallel_loop(0, _N // 16, unroll=4)
    def zero(i):
      z = jnp.zeros((16,), jnp.float32)
      for u in range(_PPT):
        accs[u][pl.ds(i * 16, 16)] = z
      if with_count:
        accC[pl.ds(i * 16, 16)] = z

    ebase = c * _EH

    def chunk(jc, carry):
      b = ebase + jc * _CHE
      pltpu.sync_copy(src_hbm.at[pl.ds(b, _CHE)], schunk)
      pltpu.sync_copy(dst_hbm.at[pl.ds(b, _CHE)], dchunk)

      @plsc.parallel_loop(0, _NST, unroll=_UNR)
      def step(i):
        svec = schunk[pl.ds(i * 16, 16)]
        dvec = dchunk[pl.ds(i * 16, 16)]
        for u in range(_PPT):
          plsc.addupdate_scatter(accs[u], [dvec],
                                 plsc.load_gather(tabs[u], [svec]))

      if with_count:
        @pl.when((s >= _NS - 2) & (jc // _QLOC == s - (_NS - 2)))
        def _():
          @plsc.parallel_loop(0, _NST, unroll=_UNR)
          def cstep(i):
            dvec = dchunk[pl.ds(i * 16, 16)]
            plsc.addupdate_scatter(accC, [dvec],
                                   jnp.full((16,), 1.0, jnp.float32))
      return carry

    lax.fori_loop(0, _NCH, chunk, 0)

    for u in range(_PPT):
      pltpu.sync_copy(accs[u],
                      agg_out.at[pl.ds((c * _HID + _PPT * s + u) * _N, _N)])
    if with_count:
      @pl.when(s >= _NS - 2)
      def _():
        q = c * 2 + (s - (_NS - 2))
        pltpu.sync_copy(accC, cnt_out.at[pl.ds(q * _N, _N)])

  return pl.kernel(
      body, out_type=out_type, mesh=mesh, scratch_types=scratch,
      compiler_params=pltpu.CompilerParams(needs_layout_passes=False))


_sc_agg_cache = {}


def _sc_agg(with_count):
  if with_count not in _sc_agg_cache:
    _sc_agg_cache[with_count] = _make_sc_agg(with_count)
  return _sc_agg_cache[with_count]


def _proj_body(x_ref, wl_ref, wr_ref, p_ref, r_ref):
  x = x_ref[...]
  dn = (((1,), (1,)), ((), ()))
  p_ref[...] = lax.dot_general(wl_ref[...], x, dn,
                               preferred_element_type=jnp.float32)
  r_ref[...] = lax.dot_general(wr_ref[...], x, dn,
                               preferred_element_type=jnp.float32)


def _l1_body(agg_ref, cnt_ref, xr_ref, b_ref, o_ref):
  cnt = cnt_ref[0] + cnt_ref[1] + cnt_ref[2] + cnt_ref[3]
  rc = 1.0 / jnp.maximum(cnt, 1.0)
  agg = agg_ref[0] + agg_ref[1]
  o_ref[...] = jnp.maximum(agg * rc + b_ref[...][:, 0:1] + xr_ref[...], 0.0)


def _l2_body(agg_ref, cnt_ref, h1_ref, wl_ref, b_ref, wr_ref, bv_ref,
             ones_ref, o_ref):
  cnt = cnt_ref[0] + cnt_ref[1] + cnt_ref[2] + cnt_ref[3]
  rc = 1.0 / jnp.maximum(cnt, 1.0)
  mean = (agg_ref[0] + agg_ref[1]) * rc
  dn = (((1,), (0,)), ((), ()))
  h2 = lax.dot_general(wl_ref[...], mean, dn,
                       preferred_element_type=jnp.float32)
  h2 = h2 + b_ref[...][:, 0:1] + lax.dot_general(
      wr_ref[...], h1_ref[...], dn, preferred_element_type=jnp.float32)
  h2 = jnp.maximum(h2, 0.0)
  oh = (bv_ref[...] == lax.broadcasted_iota(jnp.int32, (_N, _G), 1))
  oh = oh.astype(jnp.float32)
  pooled = lax.dot_general(oh, h2, (((0,), (1,)), ((), ())),
                           preferred_element_type=jnp.float32)
  cg = lax.dot_general(oh, ones_ref[...], (((0,), (0,)), ((), ())),
                       preferred_element_type=jnp.float32)
  o_ref[...] = pooled * (1.0 / jnp.maximum(cg[:, 0:1], 1.0))


_tc_proj = pl.pallas_call(
    _proj_body,
    out_shape=[jax.ShapeDtypeStruct((_HID, _N), jnp.float32),
               jax.ShapeDtypeStruct((_HID, _N), jnp.float32)])

_tc_l1 = pl.pallas_call(
    _l1_body,
    out_shape=jax.ShapeDtypeStruct((_HID, _N), jnp.float32))

_tc_l2 = pl.pallas_call(
    _l2_body,
    out_shape=jax.ShapeDtypeStruct((_G, _OUT), jnp.float32))


def kernel(x, edge_index, edge_weight, batch_vec, W1_l, b1_l, W1_r,
           W2_l, b2_l, W2_r):
  del edge_weight
  f32 = jnp.float32
  src = edge_index[0]
  dst = edge_index[1]

  p1T, xrT = _tc_proj(x, W1_l, W1_r)

  agg1T, cnt4 = _sc_agg(True)(p1T.reshape(-1), src, dst)
  agg1T = agg1T.reshape(_NC, _HID, _N)
  cnt4 = cnt4.reshape(4, _N)

  b1b = jnp.broadcast_to(b1_l[:, None], (_HID, 128)).astype(f32)
  h1T = _tc_l1(agg1T, cnt4, xrT, b1b)

  (agg2T,) = _sc_agg(False)(h1T.reshape(-1), src, dst)
  agg2T = agg2T.reshape(_NC, _HID, _N)

  b2b = jnp.broadcast_to(b2_l[:, None], (_OUT, 128)).astype(f32)
  out = _tc_l2(agg2T, cnt4, h1T, W2_l, b2b, W2_r,
               batch_vec.reshape(_N, 1), jnp.ones((_N, 8), f32))
  return out


# speedup vs baseline: 11.3781x; 1.1399x over previous
import jax
import jax.numpy as jnp
from jax import lax
from jax.experimental import pallas as pl
from jax.experimental.pallas import tpu as pltpu
from jax.experimental.pallas import tpu_sc as plsc

_N = 10000
_E = 320000
_G = 64
_HID = 64
_OUT = 128

_NC, _NS = 2, 16
_PPT = 4
_EH = _E // _NC
_CHE = 8000
_NST = _CHE // 16
_UNR = 8
_NCH = _EH // _CHE
_QLOC = _NCH // 2


def _make_sc_agg(with_count):
  mesh = plsc.VectorSubcoreMesh(core_axis_name="c", subcore_axis_name="s",
                                num_cores=_NC, num_subcores=_NS)
  out_type = [jax.ShapeDtypeStruct((_NC * _HID * _N,), jnp.float32)]
  scratch = (
      [pltpu.VMEM((_N,), jnp.float32) for _ in range(_PPT)]
      + [pltpu.VMEM((_N,), jnp.float32) for _ in range(_PPT)]
      + [pltpu.VMEM((_CHE,), jnp.int32),
         pltpu.VMEM((_CHE,), jnp.int32),
         pltpu.VMEM((_CHE,), jnp.int32),
         pltpu.VMEM((_CHE,), jnp.int32),
         pltpu.SemaphoreType.DMA, pltpu.SemaphoreType.DMA,
         pltpu.SemaphoreType.DMA, pltpu.SemaphoreType.DMA]
  )
  if with_count:
    out_type.append(jax.ShapeDtypeStruct((4 * _N,), jnp.float32))
    scratch.append(pltpu.VMEM((_N,), jnp.float32))

  def body(*refs):
    if with_count:
      (tab_hbm, src_hbm, dst_hbm, agg_out, cnt_out) = refs[:5]
      rest = refs[5:]
    else:
      (tab_hbm, src_hbm, dst_hbm, agg_out) = refs[:4]
      rest = refs[4:]
    tabs = rest[:_PPT]
    accs = rest[_PPT:2 * _PPT]
    sbufs = rest[2 * _PPT:2 * _PPT + 2], rest[2 * _PPT + 2:2 * _PPT + 4]
    sems = rest[2 * _PPT + 4:2 * _PPT + 8]
    accC = rest[2 * _PPT + 8] if with_count else None

    c = lax.axis_index("c")
    s = lax.axis_index("s")

    for u in range(_PPT):
      pltpu.sync_copy(tab_hbm.at[pl.ds((_PPT * s + u) * _N, _N)], tabs[u])

    @plsc.parallel_loop(0, _N // 16, unroll=4)
    def zero(i):
      z = jnp.zeros((16,), jnp.float32)
      for u in range(_PPT):
        accs[u][pl.ds(i * 16, 16)] = z
      if with_count:
        accC[pl.ds(i * 16, 16)] = z

    ebase = c * _EH

    def start(jc, buf):
      b = ebase + jc * _CHE
      pltpu.async_copy(src_hbm.at[pl.ds(b, _CHE)], sbufs[buf][0],
                       sems[2 * buf])
      pltpu.async_copy(dst_hbm.at[pl.ds(b, _CHE)], sbufs[buf][1],
                       sems[2 * buf + 1])

    def wait(buf):
      pltpu.make_async_copy(src_hbm.at[pl.ds(0, _CHE)], sbufs[buf][0],
                            sems[2 * buf]).wait()
      pltpu.make_async_copy(dst_hbm.at[pl.ds(0, _CHE)], sbufs[buf][1],
                            sems[2 * buf + 1]).wait()

    def process(jc, buf):
      schunk, dchunk = sbufs[buf]

      @plsc.parallel_loop(0, _NST, unroll=_UNR)
      def step(i):
        svec = schunk[pl.ds(i * 16, 16)]
        dvec = dchunk[pl.ds(i * 16, 16)]
        for u in range(_PPT):
          plsc.addupdate_scatter(accs[u], [dvec],
                                 plsc.load_gather(tabs[u], [svec]))

      if with_count:
        @pl.when((s >= _NS - 2) & (jc // _QLOC == s - (_NS - 2)))
        def _():
          @plsc.parallel_loop(0, _NST, unroll=_UNR)
          def cstep(i):
            dvec = dchunk[pl.ds(i * 16, 16)]
            plsc.addupdate_scatter(accC, [dvec],
                                   jnp.full((16,), 1.0, jnp.float32))

    start(0, 0)

    def chunk2(j2, carry):
      jc0 = j2 * 2
      start(jc0 + 1, 1)
      wait(0)
      process(jc0, 0)

      @pl.when(jc0 + 2 < _NCH)
      def _():
        start(jc0 + 2, 0)

      wait(1)
      process(jc0 + 1, 1)
      return carry

    lax.fori_loop(0, _NCH // 2, chunk2, 0)

    for u in range(_PPT):
      pltpu.sync_copy(accs[u],
                      agg_out.at[pl.ds((c * _HID + _PPT * s + u) * _N, _N)])
    if with_count:
      @pl.when(s >= _NS - 2)
      def _():
        q = c * 2 + (s - (_NS - 2))
        pltpu.sync_copy(accC, cnt_out.at[pl.ds(q * _N, _N)])

  return pl.kernel(
      body, out_type=out_type, mesh=mesh, scratch_types=scratch,
      compiler_params=pltpu.CompilerParams(needs_layout_passes=False))


_sc_agg_cache = {}


def _sc_agg(with_count):
  if with_count not in _sc_agg_cache:
    _sc_agg_cache[with_count] = _make_sc_agg(with_count)
  return _sc_agg_cache[with_count]


def _proj_body(x_ref, wl_ref, wr_ref, p_ref, r_ref):
  x = x_ref[...]
  dn = (((1,), (1,)), ((), ()))
  p_ref[...] = lax.dot_general(wl_ref[...], x, dn,
                               preferred_element_type=jnp.float32)
  r_ref[...] = lax.dot_general(wr_ref[...], x, dn,
                               preferred_element_type=jnp.float32)


def _l1_body(agg_ref, cnt_ref, xr_ref, b_ref, o_ref):
  cnt = cnt_ref[0] + cnt_ref[1] + cnt_ref[2] + cnt_ref[3]
  rc = 1.0 / jnp.maximum(cnt, 1.0)
  agg = agg_ref[0] + agg_ref[1]
  o_ref[...] = jnp.maximum(agg * rc + b_ref[...][:, 0:1] + xr_ref[...], 0.0)


def _l2_body(agg_ref, cnt_ref, h1_ref, wl_ref, b_ref, wr_ref, bv_ref,
             ones_ref, o_ref):
  cnt = cnt_ref[0] + cnt_ref[1] + cnt_ref[2] + cnt_ref[3]
  rc = 1.0 / jnp.maximum(cnt, 1.0)
  mean = (agg_ref[0] + agg_ref[1]) * rc
  dn = (((1,), (0,)), ((), ()))
  h2 = lax.dot_general(wl_ref[...], mean, dn,
                       preferred_element_type=jnp.float32)
  h2 = h2 + b_ref[...][:, 0:1] + lax.dot_general(
      wr_ref[...], h1_ref[...], dn, preferred_element_type=jnp.float32)
  h2 = jnp.maximum(h2, 0.0)
  oh = (bv_ref[...] == lax.broadcasted_iota(jnp.int32, (_N, _G), 1))
  oh = oh.astype(jnp.float32)
  pooled = lax.dot_general(oh, h2, (((0,), (1,)), ((), ())),
                           preferred_element_type=jnp.float32)
  cg = lax.dot_general(oh, ones_ref[...], (((0,), (0,)), ((), ())),
                       preferred_element_type=jnp.float32)
  o_ref[...] = pooled * (1.0 / jnp.maximum(cg[:, 0:1], 1.0))


_tc_proj = pl.pallas_call(
    _proj_body,
    out_shape=[jax.ShapeDtypeStruct((_HID, _N), jnp.float32),
               jax.ShapeDtypeStruct((_HID, _N), jnp.float32)])

_tc_l1 = pl.pallas_call(
    _l1_body,
    out_shape=jax.ShapeDtypeStruct((_HID, _N), jnp.float32))

_tc_l2 = pl.pallas_call(
    _l2_body,
    out_shape=jax.ShapeDtypeStruct((_G, _OUT), jnp.float32))


def kernel(x, edge_index, edge_weight, batch_vec, W1_l, b1_l, W1_r,
           W2_l, b2_l, W2_r):
  del edge_weight
  f32 = jnp.float32
  src = edge_index[0]
  dst = edge_index[1]

  p1T, xrT = _tc_proj(x, W1_l, W1_r)

  agg1T, cnt4 = _sc_agg(True)(p1T.reshape(-1), src, dst)
  agg1T = agg1T.reshape(_NC, _HID, _N)
  cnt4 = cnt4.reshape(4, _N)

  b1b = jnp.broadcast_to(b1_l[:, None], (_HID, 128)).astype(f32)
  h1T = _tc_l1(agg1T, cnt4, xrT, b1b)

  (agg2T,) = _sc_agg(False)(h1T.reshape(-1), src, dst)
  agg2T = agg2T.reshape(_NC, _HID, _N)

  b2b = jnp.broadcast_to(b2_l[:, None], (_OUT, 128)).astype(f32)
  out = _tc_l2(agg2T, cnt4, h1T, W2_l, b2b, W2_r,
               batch_vec.reshape(_N, 1), jnp.ones((_N, 8), f32))
  return out


# speedup vs baseline: 13.3492x; 1.1732x over previous
import jax
import jax.numpy as jnp
from jax import lax
from jax.experimental import pallas as pl
from jax.experimental.pallas import tpu as pltpu
from jax.experimental.pallas import tpu_sc as plsc

_N = 10000
_E = 320000
_G = 64
_HID = 64
_OUT = 128

_NC, _NS = 2, 16
_PPT = 4
_EH = _E // _NC
_CHE = 8000
_NST = _CHE // 16
_UNR = 10
_NCH = _EH // _CHE
_QLOC = _NCH // 2


def _make_sc_agg(with_count):
  mesh = plsc.VectorSubcoreMesh(core_axis_name="c", subcore_axis_name="s",
                                num_cores=_NC, num_subcores=_NS)
  out_type = [jax.ShapeDtypeStruct((_NC * _HID * _N,), jnp.float32)]
  scratch = (
      [pltpu.VMEM((_N,), jnp.float32) for _ in range(_PPT)]
      + [pltpu.VMEM((_N,), jnp.float32) for _ in range(_PPT)]
      + [pltpu.VMEM((_CHE,), jnp.int32),
         pltpu.VMEM((_CHE,), jnp.int32),
         pltpu.VMEM((_CHE,), jnp.int32),
         pltpu.VMEM((_CHE,), jnp.int32),
         pltpu.SemaphoreType.DMA, pltpu.SemaphoreType.DMA,
         pltpu.SemaphoreType.DMA, pltpu.SemaphoreType.DMA]
  )
  if with_count:
    out_type.append(jax.ShapeDtypeStruct((4 * _N,), jnp.float32))
    scratch.append(pltpu.VMEM((_N,), jnp.float32))

  def body(*refs):
    if with_count:
      (tab_hbm, src_hbm, dst_hbm, agg_out, cnt_out) = refs[:5]
      rest = refs[5:]
    else:
      (tab_hbm, src_hbm, dst_hbm, agg_out) = refs[:4]
      rest = refs[4:]
    tabs = rest[:_PPT]
    accs = rest[_PPT:2 * _PPT]
    sbufs = rest[2 * _PPT:2 * _PPT + 2], rest[2 * _PPT + 2:2 * _PPT + 4]
    sems = rest[2 * _PPT + 4:2 * _PPT + 8]
    accC = rest[2 * _PPT + 8] if with_count else None

    c = lax.axis_index("c")
    s = lax.axis_index("s")

    for u in range(_PPT):
      pltpu.sync_copy(tab_hbm.at[pl.ds((_PPT * s + u) * _N, _N)], tabs[u])

    @plsc.parallel_loop(0, _N // 16, unroll=4)
    def zero(i):
      z = jnp.zeros((16,), jnp.float32)
      for u in range(_PPT):
        accs[u][pl.ds(i * 16, 16)] = z
      if with_count:
        accC[pl.ds(i * 16, 16)] = z

    ebase = c * _EH

    def start(jc, buf):
      b = ebase + jc * _CHE
      pltpu.async_copy(src_hbm.at[pl.ds(b, _CHE)], sbufs[buf][0],
                       sems[2 * buf])
      pltpu.async_copy(dst_hbm.at[pl.ds(b, _CHE)], sbufs[buf][1],
                       sems[2 * buf + 1])

    def wait(buf):
      pltpu.make_async_copy(src_hbm.at[pl.ds(0, _CHE)], sbufs[buf][0],
                            sems[2 * buf]).wait()
      pltpu.make_async_copy(dst_hbm.at[pl.ds(0, _CHE)], sbufs[buf][1],
                            sems[2 * buf + 1]).wait()

    def process(jc, buf):
      schunk, dchunk = sbufs[buf]

      @plsc.parallel_loop(0, _NST, unroll=_UNR)
      def step(i):
        svec = schunk[pl.ds(i * 16, 16)]
        dvec = dchunk[pl.ds(i * 16, 16)]
        for u in range(_PPT):
          plsc.addupdate_scatter(accs[u], [dvec],
                                 plsc.load_gather(tabs[u], [svec]))

      if with_count:
        @pl.when((s >= _NS - 2) & (jc // _QLOC == s - (_NS - 2)))
        def _():
          @plsc.parallel_loop(0, _NST, unroll=_UNR)
          def cstep(i):
            dvec = dchunk[pl.ds(i * 16, 16)]
            plsc.addupdate_scatter(accC, [dvec],
                                   jnp.full((16,), 1.0, jnp.float32))

    start(0, 0)

    def chunk2(j2, carry):
      jc0 = j2 * 2
      start(jc0 + 1, 1)
      wait(0)
      process(jc0, 0)

      @pl.when(jc0 + 2 < _NCH)
      def _():
        start(jc0 + 2, 0)

      wait(1)
      process(jc0 + 1, 1)
      return carry

    lax.fori_loop(0, _NCH // 2, chunk2, 0)

    for u in range(_PPT):
      pltpu.sync_copy(accs[u],
                      agg_out.at[pl.ds((c * _HID + _PPT * s + u) * _N, _N)])
    if with_count:
      @pl.when(s >= _NS - 2)
      def _():
        q = c * 2 + (s - (_NS - 2))
        pltpu.sync_copy(accC, cnt_out.at[pl.ds(q * _N, _N)])

  return pl.kernel(
      body, out_type=out_type, mesh=mesh, scratch_types=scratch,
      compiler_params=pltpu.CompilerParams(needs_layout_passes=False))


_sc_agg_cache = {}


def _sc_agg(with_count):
  if with_count not in _sc_agg_cache:
    _sc_agg_cache[with_count] = _make_sc_agg(with_count)
  return _sc_agg_cache[with_count]


def _proj_body(x_ref, wl_ref, wr_ref, p_ref, r_ref):
  x = x_ref[...]
  dn = (((1,), (1,)), ((), ()))
  p_ref[...] = lax.dot_general(wl_ref[...], x, dn,
                               preferred_element_type=jnp.float32)
  r_ref[...] = lax.dot_general(wr_ref[...], x, dn,
                               preferred_element_type=jnp.float32)


def _l1_body(agg_ref, cnt_ref, xr_ref, b_ref, o_ref):
  cnt = cnt_ref[0] + cnt_ref[1] + cnt_ref[2] + cnt_ref[3]
  rc = 1.0 / jnp.maximum(cnt, 1.0)
  agg = agg_ref[0] + agg_ref[1]
  o_ref[...] = jnp.maximum(agg * rc + b_ref[...][:, 0:1] + xr_ref[...], 0.0)


def _l2_body(agg_ref, cnt_ref, h1_ref, wl_ref, b_ref, wr_ref, bv_ref,
             ones_ref, o_ref):
  cnt = cnt_ref[0] + cnt_ref[1] + cnt_ref[2] + cnt_ref[3]
  rc = 1.0 / jnp.maximum(cnt, 1.0)
  mean = (agg_ref[0] + agg_ref[1]) * rc
  dn = (((1,), (0,)), ((), ()))
  h2 = lax.dot_general(wl_ref[...], mean, dn,
                       preferred_element_type=jnp.float32)
  h2 = h2 + b_ref[...][:, 0:1] + lax.dot_general(
      wr_ref[...], h1_ref[...], dn, preferred_element_type=jnp.float32)
  h2 = jnp.maximum(h2, 0.0)
  oh = (bv_ref[...] == lax.broadcasted_iota(jnp.int32, (_N, _G), 1))
  oh = oh.astype(jnp.float32)
  pooled = lax.dot_general(oh, h2, (((0,), (1,)), ((), ())),
                           preferred_element_type=jnp.float32)
  cg = lax.dot_general(oh, ones_ref[...], (((0,), (0,)), ((), ())),
                       preferred_element_type=jnp.float32)
  o_ref[...] = pooled * (1.0 / jnp.maximum(cg[:, 0:1], 1.0))


_tc_proj = pl.pallas_call(
    _proj_body,
    out_shape=[jax.ShapeDtypeStruct((_HID, _N), jnp.float32),
               jax.ShapeDtypeStruct((_HID, _N), jnp.float32)])

_tc_l1 = pl.pallas_call(
    _l1_body,
    out_shape=jax.ShapeDtypeStruct((_HID, _N), jnp.float32))

_tc_l2 = pl.pallas_call(
    _l2_body,
    out_shape=jax.ShapeDtypeStruct((_G, _OUT), jnp.float32))


def kernel(x, edge_index, edge_weight, batch_vec, W1_l, b1_l, W1_r,
           W2_l, b2_l, W2_r):
  del edge_weight
  f32 = jnp.float32
  src = edge_index[0]
  dst = edge_index[1]

  p1T, xrT = _tc_proj(x, W1_l, W1_r)

  agg1T, cnt4 = _sc_agg(True)(p1T.reshape(-1), src, dst)
  agg1T = agg1T.reshape(_NC, _HID, _N)
  cnt4 = cnt4.reshape(4, _N)

  b1b = jnp.broadcast_to(b1_l[:, None], (_HID, 128)).astype(f32)
  h1T = _tc_l1(agg1T, cnt4, xrT, b1b)

  (agg2T,) = _sc_agg(False)(h1T.reshape(-1), src, dst)
  agg2T = agg2T.reshape(_NC, _HID, _N)

  b2b = jnp.broadcast_to(b2_l[:, None], (_OUT, 128)).astype(f32)
  out = _tc_l2(agg2T, cnt4, h1T, W2_l, b2b, W2_r,
               batch_vec.reshape(_N, 1), jnp.ones((_N, 8), f32))
  return out
